# Initial kernel scaffold; baseline (speedup 1.0000x reference)
import jax
import jax.numpy as jnp
from jax.experimental import pallas as pl


def kernel(x, W, pos):
    raise NotImplementedError("write your pallas kernel here")


# speedup vs baseline: 4.2141x; 4.2141x over previous
import jax
import jax.numpy as jnp
from jax import lax
from jax.experimental import pallas as pl
from jax.experimental.pallas import tpu as pltpu
from jax.experimental.pallas import tpu_sc as plsc

NC = 2
NS = 16
NW = NC * NS
C = 100
NBUF = 4
LANES = 16


def _make_body(chunks, seq, d):
    nvec = d // LANES

    def body(x_hbm, w_hbm, pos_hbm, out_hbm, idx_v, pos_v,
             buf0, buf1, buf2, buf3,
             gsem0, gsem1, gsem2, gsem3,
             wsem0, wsem1, wsem2, wsem3):
        bufs = (buf0, buf1, buf2, buf3)
        gsems = (gsem0, gsem1, gsem2, gsem3)
        wsems = (wsem0, wsem1, wsem2, wsem3)

        wid = lax.axis_index("s") * NC + lax.axis_index("c")
        pltpu.sync_copy(x_hbm.at[wid], idx_v)
        pltpu.sync_copy(pos_hbm, pos_v)

        def gather_start(k, b):
            pltpu.make_async_copy(w_hbm.at[idx_v.at[k]], bufs[b], gsems[b]).start()

        def gather_wait(b):
            pltpu.make_async_copy(w_hbm.at[idx_v.at[0]], bufs[b], gsems[b]).wait()

        def wb_start(k, b):
            pltpu.make_async_copy(bufs[b], out_hbm.at[wid, k], wsems[b]).start()

        def wb_wait(b):
            pltpu.make_async_copy(bufs[b], out_hbm.at[wid, 0], wsems[b]).wait()

        def add_pos(b, poff):
            rows = bufs[b]

            @pl.loop(0, C)
            def _(i):
                for j in range(nvec):
                    sl = pl.ds(j * LANES, LANES)
                    rows[i, sl] = rows[i, sl] + pos_v[poff + i, sl]

        gather_start(0, 0)
        gather_start(1, 1)

        @pl.loop(0, chunks, step=NBUF)
        def _(g):
            for b in range(NBUF):
                k = g + b
                bn = (b + 2) % NBUF

                @pl.when(jnp.logical_and(k >= 2, k + 2 < chunks))
                def _():
                    wb_wait(bn)

                @pl.when(k + 2 < chunks)
                def _():
                    gather_start(k + 2, bn)

                gather_wait(b)
                add_pos(b, C * (b % 2))
                wb_start(k, b)

        for b in range(NBUF):
            wb_wait(b)

    return body


def kernel(x, W, pos):
    B, S = x.shape
    V, d = W.shape
    n = B * S
    per_w = n // NW
    chunks = per_w // C
    assert n == NW * chunks * C and S == 2 * C and d % LANES == 0

    x_r = x.reshape(NW, chunks, C).astype(jnp.int32)
    mesh = plsc.VectorSubcoreMesh(
        core_axis_name="c", subcore_axis_name="s",
        num_cores=NC, num_subcores=NS)
    run = pl.kernel(
        _make_body(chunks, S, d),
        out_type=jax.ShapeDtypeStruct((NW, chunks, C, d), jnp.float32),
        mesh=mesh,
        scratch_types=[
            pltpu.VMEM((chunks, C), jnp.int32),
            pltpu.VMEM((S, d), jnp.float32),
        ] + [pltpu.VMEM((C, d), jnp.float32)] * NBUF
          + [pltpu.SemaphoreType.DMA] * (2 * NBUF),
    )
    out = run(x_r, W, pos)
    return out.reshape(B, S, d)


# speedup vs baseline: 4.2259x; 1.0028x over previous
import jax
import jax.numpy as jnp
from jax import lax
from jax.experimental import pallas as pl
from jax.experimental.pallas import tpu as pltpu
from jax.experimental.pallas import tpu_sc as plsc

NC = 2
NS = 16
NW = NC * NS
C = 100
NBUF = 4
LANES = 16


def _make_body(chunks, seq, d):
    nvec = d // LANES

    def body(x_hbm, w_hbm, pos_hbm, out_hbm, idx_v, pos_v,
             buf0, buf1, buf2, buf3,
             gsem0, gsem1, gsem2, gsem3,
             wsem0, wsem1, wsem2, wsem3):
        bufs = (buf0, buf1, buf2, buf3)
        gsems = (gsem0, gsem1, gsem2, gsem3)
        wsems = (wsem0, wsem1, wsem2, wsem3)

        wid = lax.axis_index("s") * NC + lax.axis_index("c")
        pltpu.sync_copy(x_hbm.at[wid], idx_v)
        pltpu.sync_copy(pos_hbm, pos_v)

        def gather_start(k, b):
            pltpu.make_async_copy(w_hbm.at[idx_v.at[k]], bufs[b], gsems[b]).start()

        def gather_wait(b):
            pltpu.make_async_copy(w_hbm.at[idx_v.at[0]], bufs[b], gsems[b]).wait()

        def wb_start(k, b):
            pltpu.make_async_copy(bufs[b], out_hbm.at[wid, k], wsems[b]).start()

        def wb_wait(b):
            pltpu.make_async_copy(bufs[b], out_hbm.at[wid, 0], wsems[b]).wait()

        def add_pos(b, poff):
            rows = bufs[b]

            @pl.loop(0, C, unroll=2)
            def _(i):
                for j in range(nvec):
                    sl = pl.ds(j * LANES, LANES)
                    plsc.addupdate(rows.at[i, sl], pos_v[poff + i, sl])

        gather_start(0, 0)
        gather_start(1, 1)

        @pl.loop(0, chunks, step=NBUF)
        def _(g):
            for b in range(NBUF):
                k = g + b
                bn = (b + 2) % NBUF

                @pl.when(jnp.logical_and(k >= 2, k + 2 < chunks))
                def _():
                    wb_wait(bn)

                @pl.when(k + 2 < chunks)
                def _():
                    gather_start(k + 2, bn)

                gather_wait(b)
                add_pos(b, C * (b % 2))
                wb_start(k, b)

        for b in range(NBUF):
            wb_wait(b)

    return body


def kernel(x, W, pos):
    B, S = x.shape
    V, d = W.shape
    n = B * S
    per_w = n // NW
    chunks = per_w // C
    assert n == NW * chunks * C and S == 2 * C and d % LANES == 0

    x_r = x.reshape(NW, chunks, C).astype(jnp.int32)
    mesh = plsc.VectorSubcoreMesh(
        core_axis_name="c", subcore_axis_name="s",
        num_cores=NC, num_subcores=NS)
    run = pl.kernel(
        _make_body(chunks, S, d),
        out_type=jax.ShapeDtypeStruct((NW, chunks, C, d), jnp.float32),
        mesh=mesh,
        scratch_types=[
            pltpu.VMEM((chunks, C), jnp.int32),
            pltpu.VMEM((S, d), jnp.float32),
        ] + [pltpu.VMEM((C, d), jnp.float32)] * NBUF
          + [pltpu.SemaphoreType.DMA] * (2 * NBUF),
    )
    out = run(x_r, W, pos)
    return out.reshape(B, S, d)
